# baseline (speedup 1.0000x reference)
import functools

import numpy as np

import jax
import jax.numpy as jnp
from jax import lax
from jax.experimental import pallas as pl
from jax.experimental.pallas import tpu as pltpu
from jax.experimental.pallas import tpu_sc as plsc

N = 10000
E = 320000
D_IN = 165
D_HID = 128
D_OUT = 2
NC, NS = 2, 16
NW = NC * NS
NPAD = 10240
K = 64
CHUNKS = 159
EPT = CHUNKS * K
EPW = E // NW
VECS = EPT // 16
RPT = NPAD // NS
A2F = NPAD * D_OUT

_PERM = np.array(
    [32 * (p // 16) + 2 * (p % 16) for p in range(64)]
    + [32 * (p // 16) + 2 * (p % 16) + 1 for p in range(64)], dtype=np.int32)


def _sc_mesh():
    return plsc.VectorSubcoreMesh(core_axis_name="c", subcore_axis_name="s")


def _sc_degree(ei):
    @functools.partial(
        pl.kernel,
        out_type=(
            jax.ShapeDtypeStruct((NC, NPAD), jnp.float32),
            jax.ShapeDtypeStruct((NW * EPT,), jnp.int32),
            jax.ShapeDtypeStruct((NW * EPT,), jnp.int32),
        ),
        mesh=_sc_mesh(),
        scratch_types=[
            pltpu.VMEM((EPT,), jnp.int32),
            pltpu.VMEM((EPT,), jnp.int32),
            pltpu.VMEM((NPAD,), jnp.float32),
            pltpu.VMEM((NS * RPT,), jnp.float32),
            pltpu.VMEM((RPT,), jnp.float32),
            pltpu.VMEM_SHARED((NS, NPAD), jnp.float32),
        ],
        compiler_params=pltpu.CompilerParams(
            needs_layout_passes=False, use_tc_tiling_on_sc=False),
    )
    def run(ei_h, deg_h, srcb_h, dstb_h,
            src_v, dst_v, acc_v, red_v, res_v, acc_sh):
        c = lax.axis_index("c")
        s = lax.axis_index("s")
        w = c * NS + s
        pltpu.sync_copy(ei_h.at[0, pl.ds(w * EPW, EPW)], src_v.at[pl.ds(0, EPW)])
        pltpu.sync_copy(ei_h.at[1, pl.ds(w * EPW, EPW)], dst_v.at[pl.ds(0, EPW)])
        pad16 = jnp.full((16,), NPAD - 1, jnp.int32)
        for i in range((EPT - EPW) // 16):
            src_v[pl.ds(EPW + 16 * i, 16)] = pad16
            dst_v[pl.ds(EPW + 16 * i, 16)] = pad16

        zeros16 = jnp.zeros((16,), jnp.float32)

        def zero(j, carry):
            acc_v[pl.ds(j * 16, 16)] = zeros16
            return carry

        lax.fori_loop(0, NPAD // 16, zero, 0)

        ones16 = jnp.ones((16,), jnp.float32)

        def body(j, carry):
            dv = dst_v[pl.ds(j * 16, 16)]
            plsc.addupdate_scatter(acc_v, [dv], ones16)
            return carry

        lax.fori_loop(0, VECS, body, 0)

        pltpu.sync_copy(acc_v, acc_sh.at[s])
        plsc.subcore_barrier()
        col0 = pl.multiple_of(s * RPT, 8)
        for r in range(NS):
            pltpu.sync_copy(acc_sh.at[r, pl.ds(col0, RPT)],
                            red_v.at[pl.ds(r * RPT, RPT)])

        def red(k, carry):
            v = red_v[pl.ds(k * 16, 16)]
            for r in range(1, NS):
                v = v + red_v[pl.ds(r * RPT + k * 16, 16)]
            res_v[pl.ds(k * 16, 16)] = v
            return carry

        lax.fori_loop(0, RPT // 16, red, 0)
        pltpu.sync_copy(res_v, deg_h.at[c, pl.ds(col0, RPT)])
        pltpu.sync_copy(src_v, srcb_h.at[pl.ds(w * EPT, EPT)])
        pltpu.sync_copy(dst_v, dstb_h.at[pl.ds(w * EPT, EPT)])

    return run(ei)


def _sc_aggregate(tablei, srcb, dstb, zeros_blk):
    @functools.partial(
        pl.kernel,
        out_type=(
            jax.ShapeDtypeStruct((NPAD, D_HID), jnp.float32),
            jax.ShapeDtypeStruct((NPAD, D_HID), jnp.float32),
        ),
        mesh=_sc_mesh(),
        scratch_types=[
            pltpu.VMEM((CHUNKS, K), jnp.int32),
            pltpu.VMEM((CHUNKS, K), jnp.int32),
            pltpu.VMEM((K, D_HID // 2), jnp.int32),
            pltpu.VMEM((K, D_HID // 2), jnp.int32),
            pltpu.VMEM((K, D_HID), jnp.float32),
            pltpu.VMEM((K, D_HID), jnp.float32),
            pltpu.VMEM_SHARED((NPAD, D_HID), jnp.float32),
            pltpu.SemaphoreType.DMA,
            pltpu.SemaphoreType.DMA,
            pltpu.SemaphoreType.DMA,
            pltpu.SemaphoreType.DMA,
        ],
        compiler_params=pltpu.CompilerParams(
            needs_layout_passes=False, use_tc_tiling_on_sc=False),
    )
    def run(tablei_h, srcb_h, dstb_h, zeros_h, out0_h, out1_h,
            src_v, dst_v, b16a, b16b, b32a, b32b, acc, sga, sgb, ssa, ssb):
        c = lax.axis_index("c")
        s = lax.axis_index("s")
        w = c * NS + s
        pltpu.sync_copy(srcb_h.at[w], src_v)
        pltpu.sync_copy(dstb_h.at[w], dst_v)

        pltpu.sync_copy(zeros_h, b32a)
        row0 = pl.multiple_of(s * RPT, 8)
        for i in range(RPT // K):
            pltpu.sync_copy(b32a, acc.at[pl.ds(row0 + i * K, K)])
        plsc.subcore_barrier()

        def gather(j, buf, sem):
            pltpu.async_copy(tablei_h.at[src_v.at[j]], buf, sem)

        def gather_wait(j, buf, sem):
            pltpu.make_async_copy(tablei_h.at[src_v.at[j]], buf, sem).wait()

        def scatter(j, buf, sem):
            pltpu.async_copy(buf, acc.at[dst_v.at[j]], sem, add=True)

        def scatter_wait(j, buf, sem):
            pltpu.make_async_copy(buf, acc.at[dst_v.at[j]], sem).wait()

        himask = jnp.full((16,), -65536, jnp.int32)

        def widen(b16, b32):
            def cb(t, carry):
                r = t >> 2
                m = lax.bitwise_and(t, 3)
                wv = b16[r, pl.ds(m * 16, 16)]
                b32[r, pl.ds(m * 16, 16)] = plsc.bitcast(
                    lax.shift_left(wv, 16), jnp.float32)
                b32[r, pl.ds(64 + m * 16, 16)] = plsc.bitcast(
                    lax.bitwise_and(wv, himask), jnp.float32)
                return carry

            lax.fori_loop(0, K * 4, cb, 0)

        gather(0, b16a, sga)

        def body(i, carry):
            j0 = 2 * i
            j1 = j0 + 1

            @pl.when(j1 < CHUNKS)
            def _():
                gather(j1, b16b, sgb)

            gather_wait(j0, b16a, sga)

            @pl.when(i > 0)
            def _():
                scatter_wait(j0 - 2, b32a, ssa)

            widen(b16a, b32a)

            @pl.when(j0 + 2 < CHUNKS)
            def _():
                gather(j0 + 2, b16a, sga)

            scatter(j0, b32a, ssa)

            @pl.when(j1 < CHUNKS)
            def _():
                gather_wait(j1, b16b, sgb)

                @pl.when(i > 0)
                def _():
                    scatter_wait(j1 - 2, b32b, ssb)

                widen(b16b, b32b)
                scatter(j1, b32b, ssb)

            return carry

        lax.fori_loop(0, (CHUNKS + 1) // 2, body, 0)
        scatter_wait(CHUNKS - 1, b32a, ssa)
        scatter_wait(CHUNKS - 2, b32b, ssb)
        plsc.subcore_barrier()

        @pl.when(c == 0)
        def _():
            pltpu.sync_copy(acc.at[pl.ds(row0, RPT)], out0_h.at[pl.ds(row0, RPT)])

        @pl.when(c != 0)
        def _():
            pltpu.sync_copy(acc.at[pl.ds(row0, RPT)], out1_h.at[pl.ds(row0, RPT)])

    return run(tablei, srcb, dstb, zeros_blk)


def _sc_aggregate2(g2f, srcf, dstf, zeros_flat):
    RPT2 = A2F // NS

    @functools.partial(
        pl.kernel,
        out_type=jax.ShapeDtypeStruct((NC, A2F), jnp.float32),
        mesh=_sc_mesh(),
        scratch_types=[
            pltpu.VMEM((EPT,), jnp.int32),
            pltpu.VMEM((EPT,), jnp.int32),
            pltpu.VMEM((A2F,), jnp.float32),
            pltpu.VMEM((A2F,), jnp.float32),
            pltpu.VMEM((NS * RPT2,), jnp.float32),
            pltpu.VMEM((RPT2,), jnp.float32),
            pltpu.VMEM_SHARED((NS, A2F), jnp.float32),
        ],
        compiler_params=pltpu.CompilerParams(needs_layout_passes=False),
    )
    def run(g2f_h, srcf_h, dstf_h, zeros_h, out_h,
            src_v, dst_v, tab_v, acc_v, red_v, res_v, acc_sh):
        c = lax.axis_index("c")
        s = lax.axis_index("s")
        w = c * NS + s
        pltpu.sync_copy(srcf_h.at[w], src_v)
        pltpu.sync_copy(dstf_h.at[w], dst_v)
        pltpu.sync_copy(g2f_h, tab_v)

        @pl.when(w == 0)
        def _():
            pltpu.sync_copy(g2f_h, acc_v)

        @pl.when(w != 0)
        def _():
            pltpu.sync_copy(zeros_h, acc_v)

        def body(j, carry):
            sv = src_v[pl.ds(j * 16, 16)]
            dv = dst_v[pl.ds(j * 16, 16)]
            f0s = sv * 2
            f0d = dv * 2
            v0 = plsc.load_gather(tab_v, [f0s])
            v1 = plsc.load_gather(tab_v, [f0s + 1])
            plsc.addupdate_scatter(acc_v, [f0d], v0)
            plsc.addupdate_scatter(acc_v, [f0d + 1], v1)
            return carry

        lax.fori_loop(0, VECS, body, 0)

        pltpu.sync_copy(acc_v, acc_sh.at[s])
        plsc.subcore_barrier()
        col0 = pl.multiple_of(s * RPT2, 8)
        for r in range(NS):
            pltpu.sync_copy(acc_sh.at[r, pl.ds(col0, RPT2)],
                            red_v.at[pl.ds(r * RPT2, RPT2)])

        def red(k, carry):
            v = red_v[pl.ds(k * 16, 16)]
            for r in range(1, NS):
                v = v + red_v[pl.ds(r * RPT2 + k * 16, 16)]
            res_v[pl.ds(k * 16, 16)] = v
            return carry

        lax.fori_loop(0, RPT2 // 16, red, 0)
        pltpu.sync_copy(res_v, out_h.at[c, pl.ds(col0, RPT2)])

    return run(g2f, srcf, dstf, zeros_flat)


_BM = 1024


def _tc_matmul1(xTp, W1):
    def body(xt_ref, w_ref, o_ref):
        o_ref[...] = lax.dot_general(
            xt_ref[...], w_ref[...],
            dimension_numbers=(((0,), (0,)), ((), ())),
            preferred_element_type=jnp.float32)

    return pl.pallas_call(
        body,
        grid=(NPAD // _BM,),
        in_specs=[
            pl.BlockSpec((D_IN, _BM), lambda i: (0, i)),
            pl.BlockSpec((D_IN, D_HID), lambda i: (0, 0)),
        ],
        out_specs=pl.BlockSpec((_BM, D_HID), lambda i: (i, 0)),
        out_shape=jax.ShapeDtypeStruct((NPAD, D_HID), jnp.float32),
        compiler_params=pltpu.CompilerParams(fuse_transposed_lhs_in_matmul=True),
    )(xTp, W1)


def _tc_scale(h1, degT):
    def body(h_ref, d_ref, g_ref, dv_ref):
        dinv = lax.rsqrt(jnp.sum(d_ref[...], axis=1, keepdims=True) + 1.0)
        g = h_ref[...] * dinv
        g_ref[...] = g
        dv_ref[...] = dinv

    return pl.pallas_call(
        body,
        grid=(NPAD // _BM,),
        in_specs=[
            pl.BlockSpec((_BM, D_HID), lambda i: (i, 0)),
            pl.BlockSpec((_BM, NC), lambda i: (i, 0)),
        ],
        out_specs=[
            pl.BlockSpec((_BM, D_HID), lambda i: (i, 0)),
            pl.BlockSpec((_BM, 1), lambda i: (i, 0)),
        ],
        out_shape=[
            jax.ShapeDtypeStruct((NPAD, D_HID), jnp.float32),
            jax.ShapeDtypeStruct((NPAD, 1), jnp.float32),
        ],
    )(h1, degT)


def _tc_layer2(acc0, acc1, g1, dinv, b1r, W2):
    def body(a0_ref, a1_ref, g1_ref, dv_ref, b1_ref, w2_ref, o_ref):
        i = pl.program_id(0)
        dinv = dv_ref[...]
        h1 = jnp.maximum(
            dinv * (a0_ref[...] + a1_ref[...] + g1_ref[...]) + b1_ref[...], 0.0)
        g2 = jnp.dot(h1, w2_ref[...], preferred_element_type=jnp.float32) * dinv
        rows = i * _BM + lax.broadcasted_iota(jnp.int32, (_BM, 1), 0)
        o_ref[...] = jnp.where(rows < N, g2, 0.0)

    return pl.pallas_call(
        body,
        grid=(NPAD // _BM,),
        in_specs=[
            pl.BlockSpec((_BM, D_HID), lambda i: (i, 0)),
            pl.BlockSpec((_BM, D_HID), lambda i: (i, 0)),
            pl.BlockSpec((_BM, D_HID), lambda i: (i, 0)),
            pl.BlockSpec((_BM, 1), lambda i: (i, 0)),
            pl.BlockSpec((1, D_HID), lambda i: (0, 0)),
            pl.BlockSpec((D_HID, D_OUT), lambda i: (0, 0)),
        ],
        out_specs=pl.BlockSpec((_BM, D_OUT), lambda i: (i, 0)),
        out_shape=jax.ShapeDtypeStruct((NPAD, D_OUT), jnp.float32),
    )(acc0, acc1, g1, dinv, b1r, W2)


def _tc_head(a20, a21, dinv, b2r, Wc, bcr):
    def body(a0_ref, a1_ref, dv_ref, b2_ref, wc_ref, bc_ref, o_ref):
        emb = jnp.maximum(
            dv_ref[...] * (a0_ref[...] + a1_ref[...]) + b2_ref[...], 0.0)
        z = jnp.dot(emb, wc_ref[...], preferred_element_type=jnp.float32) + bc_ref[...]
        o_ref[...] = jax.nn.sigmoid(z)

    return pl.pallas_call(
        body,
        grid=(NPAD // _BM,),
        in_specs=[
            pl.BlockSpec((_BM, D_OUT), lambda i: (i, 0)),
            pl.BlockSpec((_BM, D_OUT), lambda i: (i, 0)),
            pl.BlockSpec((_BM, 1), lambda i: (i, 0)),
            pl.BlockSpec((1, D_OUT), lambda i: (0, 0)),
            pl.BlockSpec((D_OUT, 1), lambda i: (0, 0)),
            pl.BlockSpec((1, 1), lambda i: (0, 0)),
        ],
        out_specs=pl.BlockSpec((_BM, 1), lambda i: (i, 0)),
        out_shape=jax.ShapeDtypeStruct((NPAD, 1), jnp.float32),
    )(a20, a21, dinv, b2r, Wc, bcr)


def kernel(x, edge_index, W1, b1, W2, b2, Wc, bc):
    f32 = jnp.float32
    xTp = jnp.pad(x.T, ((0, 0), (0, NPAD - N)))

    degp, srcb_flat, dstb_flat = _sc_degree(edge_index.astype(jnp.int32))
    srcb = srcb_flat.reshape(NW, CHUNKS, K)
    dstb = dstb_flat.reshape(NW, CHUNKS, K)
    srcf = srcb_flat.reshape(NW, EPT)
    dstf = dstb_flat.reshape(NW, EPT)

    h1 = _tc_matmul1(xTp, W1[:, _PERM])
    g1, dinv = _tc_scale(h1, degp.T)
    g1i = lax.bitcast_convert_type(
        g1.astype(jnp.bfloat16).reshape(NPAD, D_HID // 2, 2), jnp.int32)
    a10, a11 = _sc_aggregate(g1i, srcb, dstb, jnp.zeros((K, D_HID), f32))
    g2 = _tc_layer2(a10, a11, g1, dinv, b1[_PERM].reshape(1, D_HID), W2[_PERM, :])
    acc2 = _sc_aggregate2(g2.reshape(A2F), srcf, dstf, jnp.zeros((A2F,), f32))
    out = _tc_head(acc2[0].reshape(NPAD, D_OUT), acc2[1].reshape(NPAD, D_OUT),
                   dinv, b2.reshape(1, D_OUT), Wc, bc.reshape(1, 1))
    return out[:N]


# speedup vs baseline: 1.0597x; 1.0597x over previous
import functools

import jax
import jax.numpy as jnp
from jax import lax
from jax.experimental import pallas as pl
from jax.experimental.pallas import tpu as pltpu
from jax.experimental.pallas import tpu_sc as plsc

N = 10000
E = 320000
D_IN = 165
D_HID = 128
D_OUT = 2
NC, NS = 2, 16
NW = NC * NS
NPAD = 10240
K = 64
CHUNKS = 159
EPT = CHUNKS * K
EPW = E // NW
VECS = EPT // 16
RPT = NPAD // NS
A2F = NPAD * D_OUT


def _sc_mesh():
    return plsc.VectorSubcoreMesh(core_axis_name="c", subcore_axis_name="s")


def _sc_degree(ei):
    @functools.partial(
        pl.kernel,
        out_type=(
            jax.ShapeDtypeStruct((NC, NPAD), jnp.float32),
            jax.ShapeDtypeStruct((NW * EPT,), jnp.int32),
            jax.ShapeDtypeStruct((NW * EPT,), jnp.int32),
        ),
        mesh=_sc_mesh(),
        scratch_types=[
            pltpu.VMEM((EPT,), jnp.int32),
            pltpu.VMEM((EPT,), jnp.int32),
            pltpu.VMEM((NPAD,), jnp.float32),
            pltpu.VMEM((NS * RPT,), jnp.float32),
            pltpu.VMEM((RPT,), jnp.float32),
            pltpu.VMEM_SHARED((NS, NPAD), jnp.float32),
        ],
        compiler_params=pltpu.CompilerParams(
            needs_layout_passes=False, use_tc_tiling_on_sc=False),
    )
    def run(ei_h, deg_h, srcb_h, dstb_h,
            src_v, dst_v, acc_v, red_v, res_v, acc_sh):
        c = lax.axis_index("c")
        s = lax.axis_index("s")
        w = c * NS + s
        pltpu.sync_copy(ei_h.at[0, pl.ds(w * EPW, EPW)], src_v.at[pl.ds(0, EPW)])
        pltpu.sync_copy(ei_h.at[1, pl.ds(w * EPW, EPW)], dst_v.at[pl.ds(0, EPW)])
        pad16 = jnp.full((16,), NPAD - 1, jnp.int32)
        for i in range((EPT - EPW) // 16):
            src_v[pl.ds(EPW + 16 * i, 16)] = pad16
            dst_v[pl.ds(EPW + 16 * i, 16)] = pad16

        zeros16 = jnp.zeros((16,), jnp.float32)

        def zero(j, carry):
            acc_v[pl.ds(j * 16, 16)] = zeros16
            return carry

        lax.fori_loop(0, NPAD // 16, zero, 0)

        ones16 = jnp.ones((16,), jnp.float32)

        def body(j, carry):
            dv = dst_v[pl.ds(j * 16, 16)]
            plsc.addupdate_scatter(acc_v, [dv], ones16)
            return carry

        lax.fori_loop(0, VECS, body, 0)

        pltpu.sync_copy(acc_v, acc_sh.at[s])
        plsc.subcore_barrier()
        col0 = pl.multiple_of(s * RPT, 8)
        for r in range(NS):
            pltpu.sync_copy(acc_sh.at[r, pl.ds(col0, RPT)],
                            red_v.at[pl.ds(r * RPT, RPT)])

        def red(k, carry):
            v = red_v[pl.ds(k * 16, 16)]
            for r in range(1, NS):
                v = v + red_v[pl.ds(r * RPT + k * 16, 16)]
            res_v[pl.ds(k * 16, 16)] = v
            return carry

        lax.fori_loop(0, RPT // 16, red, 0)
        pltpu.sync_copy(res_v, deg_h.at[c, pl.ds(col0, RPT)])
        pltpu.sync_copy(src_v, srcb_h.at[pl.ds(w * EPT, EPT)])
        pltpu.sync_copy(dst_v, dstb_h.at[pl.ds(w * EPT, EPT)])

    return run(ei)


def _sc_aggregate(table, srcb, dstb, zeros_blk):
    @functools.partial(
        pl.kernel,
        out_type=(
            jax.ShapeDtypeStruct((NPAD, D_HID), jnp.float32),
            jax.ShapeDtypeStruct((NPAD, D_HID), jnp.float32),
        ),
        mesh=_sc_mesh(),
        scratch_types=[
            pltpu.VMEM((CHUNKS, K), jnp.int32),
            pltpu.VMEM((CHUNKS, K), jnp.int32),
            pltpu.VMEM((K, D_HID), jnp.float32),
            pltpu.VMEM((K, D_HID), jnp.float32),
            pltpu.VMEM((K, D_HID), jnp.float32),
            pltpu.VMEM_SHARED((NPAD, D_HID), jnp.float32),
            pltpu.SemaphoreType.DMA,
            pltpu.SemaphoreType.DMA,
            pltpu.SemaphoreType.DMA,
            pltpu.SemaphoreType.DMA,
            pltpu.SemaphoreType.DMA,
            pltpu.SemaphoreType.DMA,
        ],
        compiler_params=pltpu.CompilerParams(use_tc_tiling_on_sc=False),
    )
    def run(table_h, srcb_h, dstb_h, zeros_h, out0_h, out1_h,
            src_v, dst_v, bufa, bufb, bufc, acc,
            sga, sgb, sgc, ssa, ssb, ssc):
        c = lax.axis_index("c")
        s = lax.axis_index("s")
        w = c * NS + s
        pltpu.sync_copy(srcb_h.at[w], src_v)
        pltpu.sync_copy(dstb_h.at[w], dst_v)

        pltpu.sync_copy(zeros_h, bufa)
        row0 = pl.multiple_of(s * RPT, 8)
        for i in range((RPT + K - 1) // K):
            r = min(K, RPT - i * K)
            pltpu.sync_copy(bufa.at[pl.ds(0, r)], acc.at[pl.ds(row0 + i * K, r)])
        plsc.subcore_barrier()

        def gather(j, buf, sem):
            pltpu.async_copy(table_h.at[src_v.at[j]], buf, sem)

        def gather_wait(j, buf, sem):
            pltpu.make_async_copy(table_h.at[src_v.at[j]], buf, sem).wait()

        def scatter(j, buf, sem):
            pltpu.async_copy(buf, acc.at[dst_v.at[j]], sem, add=True)

        def scatter_wait(j, buf, sem):
            pltpu.make_async_copy(buf, acc.at[dst_v.at[j]], sem).wait()

        bufs = (bufa, bufb, bufc)
        gsems = (sga, sgb, sgc)
        ssems = (ssa, ssb, ssc)
        gather(0, bufa, sga)
        gather(1, bufb, sgb)

        def body(i, carry):
            for t in range(3):
                j = 3 * i + t
                u = (t + 2) % 3

                @pl.when(j < CHUNKS)
                def _(j=j, t=t, u=u):
                    gather_wait(j, bufs[t], gsems[t])
                    scatter(j, bufs[t], ssems[t])

                    @pl.when(j + 2 < CHUNKS)
                    def _():
                        @pl.when(j >= 1)
                        def _():
                            scatter_wait(j - 1, bufs[u], ssems[u])

                        gather(j + 2, bufs[u], gsems[u])

            return carry

        lax.fori_loop(0, (CHUNKS + 2) // 3, body, 0)
        for t in range(3):
            last = CHUNKS - 1 - (CHUNKS - 1 - t) % 3
            scatter_wait(last, bufs[t], ssems[t])
        plsc.subcore_barrier()

        @pl.when(c == 0)
        def _():
            pltpu.sync_copy(acc.at[pl.ds(row0, RPT)], out0_h.at[pl.ds(row0, RPT)])

        @pl.when(c != 0)
        def _():
            pltpu.sync_copy(acc.at[pl.ds(row0, RPT)], out1_h.at[pl.ds(row0, RPT)])

    return run(table, srcb, dstb, zeros_blk)


def _sc_aggregate2(g2f, srcf, dstf, zeros_flat):
    RPT2 = A2F // NS

    @functools.partial(
        pl.kernel,
        out_type=jax.ShapeDtypeStruct((NC, A2F), jnp.float32),
        mesh=_sc_mesh(),
        scratch_types=[
            pltpu.VMEM((EPT,), jnp.int32),
            pltpu.VMEM((EPT,), jnp.int32),
            pltpu.VMEM((A2F,), jnp.float32),
            pltpu.VMEM((A2F,), jnp.float32),
            pltpu.VMEM((NS * RPT2,), jnp.float32),
            pltpu.VMEM((RPT2,), jnp.float32),
            pltpu.VMEM_SHARED((NS, A2F), jnp.float32),
        ],
        compiler_params=pltpu.CompilerParams(needs_layout_passes=False),
    )
    def run(g2f_h, srcf_h, dstf_h, zeros_h, out_h,
            src_v, dst_v, tab_v, acc_v, red_v, res_v, acc_sh):
        c = lax.axis_index("c")
        s = lax.axis_index("s")
        w = c * NS + s
        pltpu.sync_copy(srcf_h.at[w], src_v)
        pltpu.sync_copy(dstf_h.at[w], dst_v)
        pltpu.sync_copy(g2f_h, tab_v)

        @pl.when(w == 0)
        def _():
            pltpu.sync_copy(g2f_h, acc_v)

        @pl.when(w != 0)
        def _():
            pltpu.sync_copy(zeros_h, acc_v)

        def body(j, carry):
            sv = src_v[pl.ds(j * 16, 16)]
            dv = dst_v[pl.ds(j * 16, 16)]
            f0s = sv * 2
            f0d = dv * 2
            v0 = plsc.load_gather(tab_v, [f0s])
            v1 = plsc.load_gather(tab_v, [f0s + 1])
            plsc.addupdate_scatter(acc_v, [f0d], v0)
            plsc.addupdate_scatter(acc_v, [f0d + 1], v1)
            return carry

        lax.fori_loop(0, VECS, body, 0)

        pltpu.sync_copy(acc_v, acc_sh.at[s])
        plsc.subcore_barrier()
        col0 = pl.multiple_of(s * RPT2, 8)
        for r in range(NS):
            pltpu.sync_copy(acc_sh.at[r, pl.ds(col0, RPT2)],
                            red_v.at[pl.ds(r * RPT2, RPT2)])

        def red(k, carry):
            v = red_v[pl.ds(k * 16, 16)]
            for r in range(1, NS):
                v = v + red_v[pl.ds(r * RPT2 + k * 16, 16)]
            res_v[pl.ds(k * 16, 16)] = v
            return carry

        lax.fori_loop(0, RPT2 // 16, red, 0)
        pltpu.sync_copy(res_v, out_h.at[c, pl.ds(col0, RPT2)])

    return run(g2f, srcf, dstf, zeros_flat)


_BM = 1024


def _tc_matmul1(xTp, W1):
    def body(xt_ref, w_ref, o_ref):
        o_ref[...] = lax.dot_general(
            xt_ref[...], w_ref[...],
            dimension_numbers=(((0,), (0,)), ((), ())),
            preferred_element_type=jnp.float32)

    return pl.pallas_call(
        body,
        grid=(NPAD // _BM,),
        in_specs=[
            pl.BlockSpec((D_IN, _BM), lambda i: (0, i)),
            pl.BlockSpec((D_IN, D_HID), lambda i: (0, 0)),
        ],
        out_specs=pl.BlockSpec((_BM, D_HID), lambda i: (i, 0)),
        out_shape=jax.ShapeDtypeStruct((NPAD, D_HID), jnp.float32),
        compiler_params=pltpu.CompilerParams(fuse_transposed_lhs_in_matmul=True),
    )(xTp, W1)


def _tc_scale(h1, degT):
    def body(h_ref, d_ref, g_ref, dv_ref):
        dinv = lax.rsqrt(jnp.sum(d_ref[...], axis=1, keepdims=True) + 1.0)
        g_ref[...] = h_ref[...] * dinv
        dv_ref[...] = dinv

    return pl.pallas_call(
        body,
        grid=(NPAD // _BM,),
        in_specs=[
            pl.BlockSpec((_BM, D_HID), lambda i: (i, 0)),
            pl.BlockSpec((_BM, NC), lambda i: (i, 0)),
        ],
        out_specs=[
            pl.BlockSpec((_BM, D_HID), lambda i: (i, 0)),
            pl.BlockSpec((_BM, 1), lambda i: (i, 0)),
        ],
        out_shape=[
            jax.ShapeDtypeStruct((NPAD, D_HID), jnp.float32),
            jax.ShapeDtypeStruct((NPAD, 1), jnp.float32),
        ],
    )(h1, degT)


def _tc_layer2(acc0, acc1, g1, dinv, b1r, W2):
    def body(a0_ref, a1_ref, g1_ref, dv_ref, b1_ref, w2_ref, o_ref):
        i = pl.program_id(0)
        dinv = dv_ref[...]
        h1 = jnp.maximum(
            dinv * (a0_ref[...] + a1_ref[...] + g1_ref[...]) + b1_ref[...], 0.0)
        g2 = jnp.dot(h1, w2_ref[...], preferred_element_type=jnp.float32) * dinv
        rows = i * _BM + lax.broadcasted_iota(jnp.int32, (_BM, 1), 0)
        o_ref[...] = jnp.where(rows < N, g2, 0.0)

    return pl.pallas_call(
        body,
        grid=(NPAD // _BM,),
        in_specs=[
            pl.BlockSpec((_BM, D_HID), lambda i: (i, 0)),
            pl.BlockSpec((_BM, D_HID), lambda i: (i, 0)),
            pl.BlockSpec((_BM, D_HID), lambda i: (i, 0)),
            pl.BlockSpec((_BM, 1), lambda i: (i, 0)),
            pl.BlockSpec((1, D_HID), lambda i: (0, 0)),
            pl.BlockSpec((D_HID, D_OUT), lambda i: (0, 0)),
        ],
        out_specs=pl.BlockSpec((_BM, D_OUT), lambda i: (i, 0)),
        out_shape=jax.ShapeDtypeStruct((NPAD, D_OUT), jnp.float32),
    )(acc0, acc1, g1, dinv, b1r, W2)


def _tc_head(a20, a21, dinv, b2r, Wc, bcr):
    def body(a0_ref, a1_ref, dv_ref, b2_ref, wc_ref, bc_ref, o_ref):
        emb = jnp.maximum(
            dv_ref[...] * (a0_ref[...] + a1_ref[...]) + b2_ref[...], 0.0)
        z = jnp.dot(emb, wc_ref[...], preferred_element_type=jnp.float32) + bc_ref[...]
        o_ref[...] = jax.nn.sigmoid(z)

    return pl.pallas_call(
        body,
        grid=(NPAD // _BM,),
        in_specs=[
            pl.BlockSpec((_BM, D_OUT), lambda i: (i, 0)),
            pl.BlockSpec((_BM, D_OUT), lambda i: (i, 0)),
            pl.BlockSpec((_BM, 1), lambda i: (i, 0)),
            pl.BlockSpec((1, D_OUT), lambda i: (0, 0)),
            pl.BlockSpec((D_OUT, 1), lambda i: (0, 0)),
            pl.BlockSpec((1, 1), lambda i: (0, 0)),
        ],
        out_specs=pl.BlockSpec((_BM, 1), lambda i: (i, 0)),
        out_shape=jax.ShapeDtypeStruct((NPAD, 1), jnp.float32),
    )(a20, a21, dinv, b2r, Wc, bcr)


def kernel(x, edge_index, W1, b1, W2, b2, Wc, bc):
    f32 = jnp.float32
    xTp = jnp.pad(x.T, ((0, 0), (0, NPAD - N)))

    degp, srcb_flat, dstb_flat = _sc_degree(edge_index.astype(jnp.int32))
    srcb = srcb_flat.reshape(NW, CHUNKS, K)
    dstb = dstb_flat.reshape(NW, CHUNKS, K)
    srcf = srcb_flat.reshape(NW, EPT)
    dstf = dstb_flat.reshape(NW, EPT)

    h1 = _tc_matmul1(xTp, W1)
    g1, dinv = _tc_scale(h1, degp.T)
    a10, a11 = _sc_aggregate(g1, srcb, dstb, jnp.zeros((K, D_HID), f32))
    g2 = _tc_layer2(a10, a11, g1, dinv, b1.reshape(1, D_HID), W2)
    acc2 = _sc_aggregate2(g2.reshape(A2F), srcf, dstf, jnp.zeros((A2F,), f32))
    out = _tc_head(acc2[0].reshape(NPAD, D_OUT), acc2[1].reshape(NPAD, D_OUT),
                   dinv, b2.reshape(1, D_OUT), Wc, bc.reshape(1, 1))
    return out[:N]
